# baseline (speedup 1.0000x reference)
import functools

import jax
import jax.numpy as jnp
from jax import lax
from jax.experimental import pallas as pl
from jax.experimental.pallas import tpu as pltpu
from jax.experimental.pallas import tpu_sc as plsc

_MAX_LEN = 2048
_NUM_HEADS = 16
_SEQ = 2048
_ROW = 4096
_NSHIFT = 8
_NW = 32
_ROWS_PER_W = _NUM_HEADS * _SEQ // _NW
_LANE = 128
_NT = _SEQ // _LANE


def _sc_expand(ext_flat):
    mesh = plsc.VectorSubcoreMesh(core_axis_name="c", subcore_axis_name="s")

    @functools.partial(
        pl.kernel,
        mesh=mesh,
        out_type=jax.ShapeDtypeStruct((_NUM_HEADS * _SEQ * _SEQ,), jnp.float32),
        scratch_types=[
            pltpu.VMEM((_NSHIFT * _ROW,), jnp.float32),
            pltpu.SemaphoreType.DMA,
        ],
    )
    def k(ext_hbm, out_hbm, ext_v, sem):
        wid = lax.axis_index("s") * 2 + lax.axis_index("c")
        head = wid // 2
        mbase = (wid % 2) * (_ROWS_PER_W // _NSHIFT)
        pltpu.sync_copy(
            ext_hbm.at[pl.ds(head * (_NSHIFT * _ROW), _NSHIFT * _ROW)], ext_v)

        def fire(g):
            m = mbase + g
            src0 = (_SEQ - _NSHIFT) - _NSHIFT * m
            dst0 = head * (_SEQ * _SEQ) + m * (_NSHIFT * _SEQ)
            for t in range(_NT):
                for s in range(_NSHIFT):
                    pltpu.async_copy(
                        ext_v.at[pl.ds(s * _ROW + src0 + _LANE * t, _LANE)],
                        out_hbm.at[pl.ds(dst0 + t * (_NSHIFT * _LANE)
                                         + s * _LANE, _LANE)],
                        sem,
                    )

        def drain_one_group():
            pltpu.make_async_copy(
                ext_hbm.at[pl.ds(0, _NSHIFT * _SEQ)],
                ext_v.at[pl.ds(0, _NSHIFT * _SEQ)],
                sem,
            ).wait()

        fire(0)

        def body(g, carry):
            fire(g)
            drain_one_group()
            return carry

        lax.fori_loop(1, _ROWS_PER_W // _NSHIFT, body, 0)
        drain_one_group()

    return k(ext_flat)


def _tc_build_ext(bias):
    def body(in_ref, out_ref):
        for p in range(_NSHIFT):
            for c in range(_ROW // _LANE):
                out_ref[:, p, c, :] = (
                    in_ref[:, 7 - p + _LANE * c : 135 - p + _LANE * c])

    return pl.pallas_call(
        body,
        grid=(1,),
        in_specs=[pl.BlockSpec((_NUM_HEADS, 4224), lambda i: (0, 0))],
        out_specs=pl.BlockSpec(
            (_NUM_HEADS, _NSHIFT, _ROW // _LANE, _LANE),
            lambda i: (0, 0, 0, 0)),
        out_shape=jax.ShapeDtypeStruct(
            (_NUM_HEADS, _NSHIFT, _ROW // _LANE, _LANE), jnp.float32),
    )(bias)


def kernel(seqlen, bias):
    del seqlen
    ext = _tc_build_ext(bias)
    y = _sc_expand(ext.reshape(-1))
    y = y.reshape(_NUM_HEADS, _SEQ // _NSHIFT, _NT, _NSHIFT, _LANE)
    y = y.transpose(0, 1, 3, 2, 4)
    return y.reshape(_NUM_HEADS, _SEQ, _SEQ)


# speedup vs baseline: 1.0155x; 1.0155x over previous
import functools

import jax
import jax.numpy as jnp
from jax import lax
from jax.experimental import pallas as pl
from jax.experimental.pallas import tpu as pltpu
from jax.experimental.pallas import tpu_sc as plsc

_MAX_LEN = 2048
_NUM_HEADS = 16
_SEQ = 2048
_ROW = 4096
_NSHIFT = 8
_NW = 32
_ROWS_PER_W = _NUM_HEADS * _SEQ // _NW
_LANE = 128
_NT = _SEQ // _LANE


def _sc_expand(ext_flat):
    mesh = plsc.VectorSubcoreMesh(core_axis_name="c", subcore_axis_name="s")

    @functools.partial(
        pl.kernel,
        mesh=mesh,
        out_type=jax.ShapeDtypeStruct((_NUM_HEADS * _SEQ * _SEQ,), jnp.float32),
        scratch_types=[
            pltpu.VMEM((_NSHIFT * _ROW,), jnp.float32),
            pltpu.SemaphoreType.DMA,
        ],
    )
    def k(ext_hbm, out_hbm, ext_v, sem):
        wid = lax.axis_index("s") * 2 + lax.axis_index("c")
        head = wid // 2
        mbase = (wid % 2) * (_ROWS_PER_W // _NSHIFT)
        pltpu.sync_copy(
            ext_hbm.at[pl.ds(head * (_NSHIFT * _ROW), _NSHIFT * _ROW)], ext_v)

        def fire(g):
            m = mbase + g
            src0 = (_SEQ - _NSHIFT) - _NSHIFT * m
            dst0 = head * (_SEQ * _SEQ) + m * (_NSHIFT * _SEQ)
            for t in range(_NT):
                for s in range(_NSHIFT):
                    pltpu.async_copy(
                        ext_v.at[pl.ds(s * _ROW + src0 + _LANE * t, _LANE)],
                        out_hbm.at[pl.ds(dst0 + t * (_NSHIFT * _LANE)
                                         + s * _LANE, _LANE)],
                        sem,
                    )

        def drain_one_group():
            pltpu.make_async_copy(
                ext_hbm.at[pl.ds(0, _NSHIFT * _SEQ)],
                ext_v.at[pl.ds(0, _NSHIFT * _SEQ)],
                sem,
            ).wait()

        fire(0)

        def body(g, carry):
            fire(g)
            drain_one_group()
            return carry

        lax.fori_loop(1, _ROWS_PER_W // _NSHIFT, body, 0)
        drain_one_group()

    return k(ext_flat)


def kernel(seqlen, bias):
    del seqlen
    bias_pad = jnp.pad(bias, ((0, 0), (0, _ROW + _NSHIFT - bias.shape[1])))
    ext = jnp.stack(
        [bias_pad[:, 7 - p : 7 - p + _ROW] for p in range(_NSHIFT)], axis=1
    )
    y = _sc_expand(ext.reshape(-1))
    y = y.reshape(_NUM_HEADS, _SEQ // _NSHIFT, _NT, _NSHIFT, _LANE)
    y = y.transpose(0, 1, 3, 2, 4)
    return y.reshape(_NUM_HEADS, _SEQ, _SEQ)
